# Initial kernel scaffold; baseline (speedup 1.0000x reference)
import jax
import jax.numpy as jnp
from jax.experimental import pallas as pl


def kernel(inputs, codebook):
    raise NotImplementedError("write your pallas kernel here")


# speedup vs baseline: 9.5136x; 9.5136x over previous
import functools

import jax
import jax.numpy as jnp
from jax import lax
from jax.experimental import pallas as pl
from jax.experimental.pallas import tpu as pltpu
from jax.experimental.pallas import tpu_sc as plsc

K = 8192
D = 256
N = 16384
COMMITMENT_COST = 0.25

TN = 256
GRID1 = N // TN

NC = 2
NS = 16
NW = NC * NS
BPW = N // NW
CH = 128
NCH = BPW // CH


def _k1_body(x_ref, cb_ref, idx_ref, counts_ref, c2_ref):
    i = pl.program_id(0)

    @pl.when(i == 0)
    def _init():
        cb = cb_ref[...]
        c2_ref[...] = jnp.sum(cb * cb, axis=1)[None, :]

    x = x_ref[...]
    mm = lax.dot_general(
        x.astype(jnp.bfloat16), cb_ref[...].astype(jnp.bfloat16),
        dimension_numbers=(((1,), (1,)), ((), ())),
        preferred_element_type=jnp.float32)
    x2 = jnp.sum(x * x, axis=1, keepdims=True)
    dist = (x2 + c2_ref[...]) - 2.0 * mm
    idx = jnp.argmin(dist, axis=1).astype(jnp.int32)
    idx_ref[...] = idx[None, None, :]

    iota = lax.broadcasted_iota(jnp.int32, (TN, K), 1)
    onehot = jnp.where(idx[:, None] == iota, 1.0, 0.0)
    cnt = jnp.sum(onehot, axis=0)[None, :]

    @pl.when(i == 0)
    def _set():
        counts_ref[...] = cnt

    @pl.when(i > 0)
    def _acc():
        counts_ref[...] += cnt


def _k1(xt, codebook):
    return pl.pallas_call(
        _k1_body,
        grid=(GRID1,),
        in_specs=[
            pl.BlockSpec((TN, D), lambda i: (i, 0)),
            pl.BlockSpec((K, D), lambda i: (0, 0)),
        ],
        out_specs=[
            pl.BlockSpec((1, 1, TN), lambda i: (i, 0, 0)),
            pl.BlockSpec((1, K), lambda i: (0, 0)),
        ],
        out_shape=[
            jax.ShapeDtypeStruct((GRID1, 1, TN), jnp.int32),
            jax.ShapeDtypeStruct((1, K), jnp.float32),
        ],
        scratch_shapes=[pltpu.VMEM((1, K), jnp.float32)],
    )(xt, codebook)


def _k2_sc_body(table_hbm, idx_hbm, out_hbm, idx_v, rows0, rows1, sem0, sem1):
    wid = lax.axis_index("s") * NC + lax.axis_index("c")
    base = wid * BPW
    pltpu.sync_copy(idx_hbm.at[pl.ds(NCH * wid, NCH)], idx_v)
    rows = (rows0, rows1)
    sems = (sem0, sem1)
    cps = [None, None]
    for c in range(NCH):
        b = c % 2
        if cps[b] is not None:
            cps[b].wait()
            pltpu.sync_copy(rows[b], out_hbm.at[pl.ds(base + (c - 2) * CH, CH)])
        cps[b] = pltpu.async_copy(table_hbm.at[idx_v.at[c]], rows[b], sems[b])
    for c in range(NCH - 2, NCH):
        b = c % 2
        cps[b].wait()
        pltpu.sync_copy(rows[b], out_hbm.at[pl.ds(base + c * CH, CH)])


def _k2_sc(codebook, idx2d):
    mesh = plsc.VectorSubcoreMesh(core_axis_name="c", subcore_axis_name="s")
    fn = functools.partial(
        pl.kernel,
        mesh=mesh,
        out_type=jax.ShapeDtypeStruct((N, D), jnp.float32),
        scratch_types=[
            pltpu.VMEM((NCH, CH), jnp.int32),
            pltpu.VMEM((CH, D), jnp.float32),
            pltpu.VMEM((CH, D), jnp.float32),
            pltpu.SemaphoreType.DMA,
            pltpu.SemaphoreType.DMA,
        ],
    )(_k2_sc_body)
    return fn(codebook, idx2d)


def _k3_body(x_ref, q_ref, counts_ref, loss_ref, perp_ref, acc_ref):
    i = pl.program_id(0)
    d = q_ref[...] - x_ref[...]
    s = jnp.sum(d * d)

    @pl.when(i == 0)
    def _set():
        acc_ref[0, 0] = s

    @pl.when(i > 0)
    def _acc():
        acc_ref[0, 0] += s

    @pl.when(i == pl.num_programs(0) - 1)
    def _fin():
        loss = COMMITMENT_COST * (acc_ref[0, 0] / float(N * D))
        loss_ref[...] = jnp.full((1, 1), loss, jnp.float32)
        avg = counts_ref[...] / float(N)
        ent = -jnp.sum(avg * jnp.log(avg + 1e-10))
        perp_ref[...] = jnp.full((1, 1), jnp.exp(ent), jnp.float32)


def _k3(xt, q, counts):
    tb = 1024
    return pl.pallas_call(
        _k3_body,
        grid=(N // tb,),
        in_specs=[
            pl.BlockSpec((tb, D), lambda i: (i, 0)),
            pl.BlockSpec((tb, D), lambda i: (i, 0)),
            pl.BlockSpec((1, K), lambda i: (0, 0)),
        ],
        out_specs=[
            pl.BlockSpec((1, 1), lambda i: (0, 0)),
            pl.BlockSpec((1, 1), lambda i: (0, 0)),
        ],
        out_shape=[
            jax.ShapeDtypeStruct((1, 1), jnp.float32),
            jax.ShapeDtypeStruct((1, 1), jnp.float32),
        ],
        scratch_shapes=[pltpu.SMEM((1, 1), jnp.float32)],
    )(xt, q, counts)


def kernel(inputs, codebook):
    xt = jnp.transpose(inputs, (0, 2, 3, 1)).reshape(N, D)
    idx3, counts = _k1(xt, codebook)
    idx = idx3.reshape(N)
    q = _k2_sc(codebook, idx.reshape(NW * NCH, CH))
    loss11, perp11 = _k3(xt, q, counts)
    quantized_out = jnp.transpose(
        q.reshape(inputs.shape[0], 32, 32, D), (0, 3, 1, 2))
    return (loss11.reshape(()), quantized_out, perp11.reshape(()),
            idx.reshape(N, 1))
